# baseline (speedup 1.0000x reference)
import functools

import jax
import jax.numpy as jnp
from jax import lax
from jax.experimental import pallas as pl
from jax.experimental.pallas import tpu as pltpu
from jax.experimental.pallas import tpu_sc as plsc

N = 50000
D = 128
K = 16
R = 4
G_ = 2 * R
BR = 10000
ITERS = 2

NP_ = 50176
NENT = 20480
NW = 32
EPW = NENT // NW
ICH = 128

CLS_BLK = 1024
REL_BLK = 1000


def _class_update_body(e_ref, m_ref, va_ref, vb_ref, wa_ref, wb_ref, o_ref):
    e = e_ref[...]
    m = m_ref[...]
    dn = (((1,), (1,)), ((), ()))
    gate_pre = (jax.lax.dot_general(e, va_ref[...], dn)
                + jax.lax.dot_general(m, vb_ref[...], dn))
    dir_pre = (jax.lax.dot_general(e, wa_ref[...], dn)
               + jax.lax.dot_general(m, wb_ref[...], dn))
    x = e + jax.nn.sigmoid(gate_pre) * jnp.maximum(dir_pre, 0.0)
    n = jnp.sqrt(jnp.sum(x * x, axis=1, keepdims=True))
    o_ref[...] = x / jnp.maximum(n, 1e-12)


def _class_update(e, m, class_V, class_W):
    va, vb = class_V[:, :D], class_V[:, D:]
    wa, wb = class_W[:, :D], class_W[:, D:]
    full = lambda s: pl.BlockSpec(s, lambda i: (0, 0))
    return pl.pallas_call(
        _class_update_body,
        grid=(NP_ // CLS_BLK,),
        in_specs=[
            pl.BlockSpec((CLS_BLK, D), lambda i: (i, 0)),
            pl.BlockSpec((CLS_BLK, K), lambda i: (i, 0)),
            full((D, D)), full((D, K)), full((D, D)), full((D, K)),
        ],
        out_specs=pl.BlockSpec((CLS_BLK, D), lambda i: (i, 0)),
        out_shape=jax.ShapeDtypeStruct((NP_, D), jnp.float32),
    )(e, m, va, vb, wa, wb)


def _rel_body(es_ref, eo_ref, svs_ref, svo_ref, sws_ref, swo_ref, sw_ref,
              ovs_ref, ovo_ref, ows_ref, owo_ref, ow_ref, o_ref):
    es = es_ref[...]
    eo = eo_ref[...]
    dn = (((1,), (1,)), ((), ()))
    dg = jax.lax.dot_general
    gs = jax.nn.sigmoid(dg(es, svs_ref[...], dn) + dg(eo, svo_ref[...], dn))
    dot_s = jnp.sum(eo * sw_ref[...], axis=1, keepdims=True)
    dir_s = jnp.maximum(dg(es, sws_ref[...], dn) + dg(eo, swo_ref[...], dn)
                        + es * dot_s, 0.0)
    o_ref[0] = gs * dir_s
    go = jax.nn.sigmoid(dg(es, ovs_ref[...], dn) + dg(eo, ovo_ref[...], dn))
    dot_o = jnp.sum(eo * ow_ref[...], axis=1, keepdims=True)
    dir_o = jnp.maximum(dg(es, ows_ref[...], dn) + dg(eo, owo_ref[...], dn)
                        + es * dot_o, 0.0)
    o_ref[1] = go * dir_o


HALF = NENT // 2
RBLK = 1024


def _rel_compute(rows, svs, svo, sws, swo, sw, ovs, ovo, ows, owo, ow):
    full = lambda: pl.BlockSpec((D, D), lambda i: (0, 0))
    vec = lambda: pl.BlockSpec((1, D), lambda i: (0, 0))
    nblk = HALF // RBLK
    return pl.pallas_call(
        _rel_body,
        grid=(nblk,),
        in_specs=[pl.BlockSpec((RBLK, D), lambda i: (i, 0)),
                  pl.BlockSpec((RBLK, D), lambda i, n=nblk: (i + n, 0)),
                  full(), full(), full(), full(), vec(),
                  full(), full(), full(), full(), vec()],
        out_specs=pl.BlockSpec((2, RBLK, D), lambda i: (0, i, 0)),
        out_shape=jax.ShapeDtypeStruct((2, HALF, D), jnp.float32),
    )(rows, rows, svs, svo, sws, swo, sw.reshape(1, D),
      ovs, ovo, ows, owo, ow.reshape(1, D))


def _norm_rows_body(x_ref, o_ref):
    x = x_ref[...]
    n = jnp.sqrt(jnp.sum(x * x, axis=1, keepdims=True))
    o_ref[...] = x / jnp.maximum(n, 1e-12)


def _norm_rows(x):
    nb = 1024 if x.shape[0] % 1024 == 0 else 64
    blk = pl.BlockSpec((nb, D), lambda i: (i, 0))
    return pl.pallas_call(
        _norm_rows_body,
        grid=(x.shape[0] // nb,),
        in_specs=[blk],
        out_specs=blk,
        out_shape=jax.ShapeDtypeStruct(x.shape, jnp.float32),
    )(x)


def _sc_gather(e, idx3d):
    kpw = NENT // ICH // NW
    mesh = plsc.VectorSubcoreMesh(core_axis_name="c", subcore_axis_name="s")

    @functools.partial(
        pl.kernel, mesh=mesh,
        out_type=jax.ShapeDtypeStruct((NENT, D), jnp.float32),
        scratch_types=[
            pltpu.VMEM((kpw, ICH), jnp.int32),
            pltpu.VMEM((EPW, D), jnp.float32),
            pltpu.SemaphoreType.DMA,
        ],
        name="sc_gather",
    )
    def k(e_hbm, idx_hbm, out_hbm, idx_v, rows_v, sem):
        wid = lax.axis_index("s") * 2 + lax.axis_index("c")
        pltpu.sync_copy(idx_hbm.at[wid], idx_v)
        cps = []
        for j in range(kpw):
            cps.append(pltpu.async_copy(
                e_hbm.at[idx_v.at[j]],
                rows_v.at[pl.ds(j * ICH, ICH)], sem))
        for cp in cps:
            cp.wait()
        pltpu.sync_copy(rows_v, out_hbm.at[pl.ds(wid * EPW, EPW)])

    return k(e, idx3d)


QROWS = 8384
NPASS = 3
GSLOT = 64
ACC_ROWS = QROWS + GSLOT
NCH = NENT // 16 // ICH


def _sc_scatter_acc(rows, upd, idx16):
    mesh = plsc.VectorSubcoreMesh(core_axis_name="c", subcore_axis_name="s")

    @functools.partial(
        pl.kernel, mesh=mesh,
        out_type=jax.ShapeDtypeStruct((NENT + GSLOT, D), jnp.float32),
        scratch_types=[
            pltpu.VMEM((NCH, ICH), jnp.int32),
            pltpu.VMEM((NCH, ICH), jnp.int32),
            pltpu.VMEM((NCH, ICH), jnp.int32),
            pltpu.VMEM((2, ICH, D), jnp.float32),
            pltpu.VMEM_SHARED((ACC_ROWS, D), jnp.float32),
            pltpu.SemaphoreType.DMA,
            pltpu.SemaphoreType.DMA,
        ],
        name="sc_scatter_acc",
    )
    def k(rows_hbm, upd_hbm, idx_hbm, accv_hbm, idx_v, aidx_v, uidx_v,
          buf_v, acc, sem_ld, sem_st):
        c = lax.axis_index("c")
        s = lax.axis_index("s")
        lane = lax.iota(jnp.int32, 16)
        pltpu.sync_copy(idx_hbm.at[s], idx_v)
        for p in range(NPASS):
            qb = (2 * p) * QROWS + c * QROWS
            for kk in range(NCH):
                for v in range(8):
                    sl = pl.ds(v * 16, 16)
                    iv = idx_v[kk, sl]
                    rel = iv - qb
                    inq = (rel >= 0) & (rel < QROWS)
                    spread = (lane + (kk * 8 + v)) & (GSLOT - 1)
                    aidx_v[kk, sl] = jnp.where(inq, rel, QROWS + spread)
                    ebase = s * (NCH * ICH) + kk * ICH + v * 16
                    uidx_v[kk, sl] = jnp.where(inq, ebase + lane, NENT + spread)
            cps = []
            for kk in range(NCH):
                if kk >= 2:
                    cps[kk - 2].wait()
                off = pl.multiple_of((s * NCH + kk) * ICH, ICH)
                pltpu.sync_copy(rows_hbm.at[pl.ds(off, ICH)], buf_v.at[kk % 2])
                cps.append(pltpu.async_copy(
                    buf_v.at[kk % 2], acc.at[aidx_v.at[kk]], sem_st))
            for cp in cps[-2:]:
                cp.wait()
            plsc.subcore_barrier()
            cps = []
            for kk in range(NCH):
                if kk >= 2:
                    cps[kk - 2].wait()
                off = pl.multiple_of((s * NCH + kk) * ICH, ICH)
                pltpu.sync_copy(upd_hbm.at[pl.ds(off, ICH)], buf_v.at[kk % 2])
                cps.append(pltpu.async_copy(
                    buf_v.at[kk % 2], acc.at[aidx_v.at[kk]], sem_st, add=True))
            for cp in cps[-2:]:
                cp.wait()
            plsc.subcore_barrier()
            cps = []
            for kk in range(NCH):
                if kk >= 2:
                    cps[kk - 2].wait()
                pltpu.async_copy(acc.at[aidx_v.at[kk]], buf_v.at[kk % 2],
                                 sem_ld).wait()
                cps.append(pltpu.async_copy(
                    buf_v.at[kk % 2], accv_hbm.at[uidx_v.at[kk]], sem_st))
            for cp in cps[-2:]:
                cp.wait()
            if p < NPASS - 1:
                plsc.subcore_barrier()

    return k(rows, upd, idx16)


def _sc_scatter_rows(e_ref, idx3d_g, un):
    kpw = NENT // ICH // NW
    mesh = plsc.VectorSubcoreMesh(core_axis_name="c", subcore_axis_name="s")

    @functools.partial(
        pl.kernel, mesh=mesh, out_type=(),
        scratch_types=[
            pltpu.VMEM((kpw, ICH), jnp.int32),
            pltpu.VMEM((2, ICH, D), jnp.float32),
            pltpu.SemaphoreType.DMA,
        ],
        name="sc_scatter_rows",
    )
    def k(e_hbm, idx_hbm, un_hbm, idx_v, un_v, sem):
        wid = lax.axis_index("s") * 2 + lax.axis_index("c")
        pltpu.sync_copy(idx_hbm.at[wid], idx_v)
        cps = []
        for j in range(kpw):
            if j >= 2:
                cps[j - 2].wait()
            off = pl.multiple_of((wid * kpw + j) * ICH, ICH)
            pltpu.sync_copy(un_hbm.at[pl.ds(off, ICH)], un_v.at[j % 2])
            cps.append(pltpu.async_copy(un_v.at[j % 2],
                                        e_hbm.at[idx_v.at[j]], sem))
        for cp in cps[-2:]:
            cp.wait()

    k(e_ref, idx3d_g, un)


def kernel(embeddings, memberships, subjects, objects, class_V, class_W,
           sub_Vs, sub_Vo, sub_Ws, sub_Wo, sub_w,
           obj_Vs, obj_Vo, obj_Ws, obj_Wo, obj_w):
    e = jnp.pad(embeddings, ((0, NP_ - N), (0, 0)))
    m_pad = jnp.pad(memberships, ((0, NP_ - N), (0, 0)))
    npad = (NENT - 2 * BR) // 2
    pad_s = N + (jnp.arange(npad, dtype=jnp.int32) % (NP_ - N))
    pad_o = N + ((jnp.arange(npad, dtype=jnp.int32) + npad) % (NP_ - N))
    idx_all = jnp.concatenate(
        [subjects, jnp.tile(pad_s, (G_, 1)),
         objects, jnp.tile(pad_o, (G_, 1))], axis=1)
    idx3d = idx_all.reshape(G_, NW, NENT // ICH // NW, ICH)
    idx16 = idx_all.reshape(G_, 16, NCH, ICH)

    for _ in range(ITERS):
        e = _class_update(e, m_pad, class_V, class_W)
        e_ref = jax.new_ref(e)
        rows = _sc_gather(e_ref, idx3d[0])
        for g in range(G_):
            upd = _rel_compute(
                rows,
                sub_Vs[g], sub_Vo[g], sub_Ws[g], sub_Wo[g], sub_w[g],
                obj_Vs[g], obj_Vo[g], obj_Ws[g], obj_Wo[g], obj_w[g]
            ).reshape(NENT, D)
            accv = _sc_scatter_acc(rows, upd, idx16[g])
            un = _norm_rows(accv)
            _sc_scatter_rows(e_ref, idx3d[g], un)
            if g < G_ - 1:
                rows = _sc_gather(e_ref, idx3d[g + 1])
        e = e_ref[...]
    return e[:N]


# speedup vs baseline: 1.5095x; 1.5095x over previous
import functools

import jax
import jax.numpy as jnp
from jax import lax
from jax.experimental import pallas as pl
from jax.experimental.pallas import tpu as pltpu
from jax.experimental.pallas import tpu_sc as plsc

N = 50000
D = 128
K = 16
R = 4
G_ = 2 * R
BR = 10000
ITERS = 2

NP_ = 50176
NENT = 20480
NW = 32
EPW = NENT // NW
ICH = 128

CLS_BLK = 1024
REL_BLK = 1000


def _class_update_body(e_ref, m_ref, va_ref, vb_ref, wa_ref, wb_ref, o_ref):
    e = e_ref[...]
    m = m_ref[...]
    dn = (((1,), (1,)), ((), ()))
    gate_pre = (jax.lax.dot_general(e, va_ref[...], dn)
                + jax.lax.dot_general(m, vb_ref[...], dn))
    dir_pre = (jax.lax.dot_general(e, wa_ref[...], dn)
               + jax.lax.dot_general(m, wb_ref[...], dn))
    x = e + jax.nn.sigmoid(gate_pre) * jnp.maximum(dir_pre, 0.0)
    n = jnp.sqrt(jnp.sum(x * x, axis=1, keepdims=True))
    o_ref[...] = x / jnp.maximum(n, 1e-12)


def _class_update(e, m, class_V, class_W):
    va, vb = class_V[:, :D], class_V[:, D:]
    wa, wb = class_W[:, :D], class_W[:, D:]
    full = lambda s: pl.BlockSpec(s, lambda i: (0, 0))
    return pl.pallas_call(
        _class_update_body,
        grid=(NP_ // CLS_BLK,),
        in_specs=[
            pl.BlockSpec((CLS_BLK, D), lambda i: (i, 0)),
            pl.BlockSpec((CLS_BLK, K), lambda i: (i, 0)),
            full((D, D)), full((D, K)), full((D, D)), full((D, K)),
        ],
        out_specs=pl.BlockSpec((CLS_BLK, D), lambda i: (i, 0)),
        out_shape=jax.ShapeDtypeStruct((NP_, D), jnp.float32),
    )(e, m, va, vb, wa, wb)


def _rel_body(es_ref, eo_ref, svs_ref, svo_ref, sws_ref, swo_ref, sw_ref,
              ovs_ref, ovo_ref, ows_ref, owo_ref, ow_ref, o_ref):
    es = es_ref[...]
    eo = eo_ref[...]
    dn = (((1,), (1,)), ((), ()))
    dg = jax.lax.dot_general
    gs = jax.nn.sigmoid(dg(es, svs_ref[...], dn) + dg(eo, svo_ref[...], dn))
    dot_s = jnp.sum(eo * sw_ref[...], axis=1, keepdims=True)
    dir_s = jnp.maximum(dg(es, sws_ref[...], dn) + dg(eo, swo_ref[...], dn)
                        + es * dot_s, 0.0)
    o_ref[0] = gs * dir_s
    go = jax.nn.sigmoid(dg(es, ovs_ref[...], dn) + dg(eo, ovo_ref[...], dn))
    dot_o = jnp.sum(eo * ow_ref[...], axis=1, keepdims=True)
    dir_o = jnp.maximum(dg(es, ows_ref[...], dn) + dg(eo, owo_ref[...], dn)
                        + es * dot_o, 0.0)
    o_ref[1] = go * dir_o


HALF = NENT // 2
RBLK = 1024


def _rel_compute(rows, svs, svo, sws, swo, sw, ovs, ovo, ows, owo, ow):
    full = lambda: pl.BlockSpec((D, D), lambda i: (0, 0))
    vec = lambda: pl.BlockSpec((1, D), lambda i: (0, 0))
    nblk = HALF // RBLK
    return pl.pallas_call(
        _rel_body,
        grid=(nblk,),
        in_specs=[pl.BlockSpec((RBLK, D), lambda i: (i, 0)),
                  pl.BlockSpec((RBLK, D), lambda i, n=nblk: (i + n, 0)),
                  full(), full(), full(), full(), vec(),
                  full(), full(), full(), full(), vec()],
        out_specs=pl.BlockSpec((2, RBLK, D), lambda i: (0, i, 0)),
        out_shape=jax.ShapeDtypeStruct((2, HALF, D), jnp.float32),
    )(rows, rows, svs, svo, sws, swo, sw.reshape(1, D),
      ovs, ovo, ows, owo, ow.reshape(1, D))


def _norm_add_body(r_ref, a_ref, o_ref):
    x = r_ref[...] + a_ref[...]
    n = jnp.sqrt(jnp.sum(x * x, axis=1, keepdims=True))
    o_ref[...] = x / jnp.maximum(n, 1e-12)


def _norm_add(rows, accv):
    nb = 1024
    blk = pl.BlockSpec((nb, D), lambda i: (i, 0))
    return pl.pallas_call(
        _norm_add_body,
        grid=(NENT // nb,),
        in_specs=[blk, blk],
        out_specs=blk,
        out_shape=jax.ShapeDtypeStruct((NENT, D), jnp.float32),
    )(rows, accv)


def _sc_gather(e, idx3d):
    kpw = NENT // ICH // NW
    mesh = plsc.VectorSubcoreMesh(core_axis_name="c", subcore_axis_name="s")

    @functools.partial(
        pl.kernel, mesh=mesh,
        out_type=jax.ShapeDtypeStruct((NENT, D), jnp.float32),
        scratch_types=[
            pltpu.VMEM((kpw, ICH), jnp.int32),
            pltpu.VMEM((EPW, D), jnp.float32),
            pltpu.SemaphoreType.DMA,
        ],
        name="sc_gather",
    )
    def k(e_hbm, idx_hbm, out_hbm, idx_v, rows_v, sem):
        wid = lax.axis_index("s") * 2 + lax.axis_index("c")
        pltpu.sync_copy(idx_hbm.at[wid], idx_v)
        cps = []
        for j in range(kpw):
            cps.append(pltpu.async_copy(
                e_hbm.at[idx_v.at[j]],
                rows_v.at[pl.ds(j * ICH, ICH)], sem))
        for cp in cps:
            cp.wait()
        pltpu.sync_copy(rows_v, out_hbm.at[pl.ds(wid * EPW, EPW)])

    return k(e, idx3d)


QROWS = 8384
NPASS = 3
GSLOT = 64
ACC_ROWS = QROWS + GSLOT
NCH = NENT // 16 // ICH


def _sc_scatter_acc(upd, idx16, zeros):
    mesh = plsc.VectorSubcoreMesh(core_axis_name="c", subcore_axis_name="s")

    @functools.partial(
        pl.kernel, mesh=mesh,
        out_type=jax.ShapeDtypeStruct((NENT + GSLOT, D), jnp.float32),
        scratch_types=[
            pltpu.VMEM((NCH, ICH), jnp.int32),
            pltpu.VMEM((NCH, ICH), jnp.int32),
            pltpu.VMEM((NCH, ICH), jnp.int32),
            pltpu.VMEM((2, ICH, D), jnp.float32),
            pltpu.VMEM((ICH, D), jnp.float32),
            pltpu.VMEM_SHARED((ACC_ROWS, D), jnp.float32),
            pltpu.SemaphoreType.DMA,
            pltpu.SemaphoreType.DMA,
        ],
        name="sc_scatter_acc",
    )
    def k(upd_hbm, idx_hbm, z_hbm, accv_hbm, idx_v, aidx_v, uidx_v,
          buf_v, zeros_v, acc, sem_ld, sem_st):
        c = lax.axis_index("c")
        s = lax.axis_index("s")
        lane = lax.iota(jnp.int32, 16)
        pltpu.sync_copy(idx_hbm.at[s], idx_v)
        pltpu.sync_copy(z_hbm, zeros_v)
        stripe = ACC_ROWS // 16
        nfull = stripe // ICH
        for j in range(nfull):
            pltpu.sync_copy(zeros_v, acc.at[pl.ds(s * stripe + j * ICH, ICH)])
        rem = stripe - nfull * ICH
        if rem:
            pltpu.sync_copy(zeros_v.at[pl.ds(0, rem)],
                            acc.at[pl.ds(s * stripe + nfull * ICH, rem)])
        plsc.subcore_barrier()
        for p in range(NPASS):
            qb = (2 * p) * QROWS + c * QROWS
            for kk in range(NCH):
                for v in range(8):
                    sl = pl.ds(v * 16, 16)
                    iv = idx_v[kk, sl]
                    rel = iv - qb
                    inq = (rel >= 0) & (rel < QROWS)
                    spread = (lane + (kk * 8 + v)) & (GSLOT - 1)
                    aidx_v[kk, sl] = jnp.where(inq, rel, QROWS + spread)
                    ebase = s * (NCH * ICH) + kk * ICH + v * 16
                    uidx_v[kk, sl] = jnp.where(inq, ebase + lane, NENT + spread)
            lds = [None] * NCH
            sts = []
            off0 = pl.multiple_of(s * NCH * ICH, ICH)
            lds[0] = pltpu.async_copy(upd_hbm.at[pl.ds(off0, ICH)],
                                      buf_v.at[0], sem_ld)
            for kk in range(NCH):
                if kk + 1 < NCH:
                    if kk >= 1:
                        sts[kk - 1].wait()
                    off = pl.multiple_of((s * NCH + kk + 1) * ICH, ICH)
                    lds[kk + 1] = pltpu.async_copy(
                        upd_hbm.at[pl.ds(off, ICH)],
                        buf_v.at[(kk + 1) % 2], sem_ld)
                lds[kk].wait()
                sts.append(pltpu.async_copy(
                    buf_v.at[kk % 2], acc.at[aidx_v.at[kk]], sem_st, add=True))
            sts[-2].wait()
            sts[-1].wait()
            plsc.subcore_barrier()
            gts = [None] * NCH
            sts = []
            gts[0] = pltpu.async_copy(acc.at[aidx_v.at[0]], buf_v.at[0], sem_ld)
            for kk in range(NCH):
                if kk + 1 < NCH:
                    if kk >= 1:
                        sts[kk - 1].wait()
                    gts[kk + 1] = pltpu.async_copy(
                        acc.at[aidx_v.at[kk + 1]], buf_v.at[(kk + 1) % 2],
                        sem_ld)
                gts[kk].wait()
                sts.append(pltpu.async_copy(
                    buf_v.at[kk % 2], accv_hbm.at[uidx_v.at[kk]], sem_st))
            sts[-2].wait()
            sts[-1].wait()
            plsc.subcore_barrier()
            if p < NPASS - 1:
                zs = [pltpu.async_copy(zeros_v, acc.at[aidx_v.at[kk]], sem_st)
                      for kk in range(NCH)]
                for z in zs:
                    z.wait()
                plsc.subcore_barrier()

    return k(upd, idx16, zeros)


def _sc_scatter_rows(e_ref, idx3d_g, un):
    kpw = NENT // ICH // NW
    mesh = plsc.VectorSubcoreMesh(core_axis_name="c", subcore_axis_name="s")

    @functools.partial(
        pl.kernel, mesh=mesh, out_type=(),
        scratch_types=[
            pltpu.VMEM((kpw, ICH), jnp.int32),
            pltpu.VMEM((2, ICH, D), jnp.float32),
            pltpu.SemaphoreType.DMA,
        ],
        name="sc_scatter_rows",
    )
    def k(e_hbm, idx_hbm, un_hbm, idx_v, un_v, sem):
        wid = lax.axis_index("s") * 2 + lax.axis_index("c")
        pltpu.sync_copy(idx_hbm.at[wid], idx_v)
        cps = []
        for j in range(kpw):
            if j >= 2:
                cps[j - 2].wait()
            off = pl.multiple_of((wid * kpw + j) * ICH, ICH)
            pltpu.sync_copy(un_hbm.at[pl.ds(off, ICH)], un_v.at[j % 2])
            cps.append(pltpu.async_copy(un_v.at[j % 2],
                                        e_hbm.at[idx_v.at[j]], sem))
        for cp in cps[-2:]:
            cp.wait()

    k(e_ref, idx3d_g, un)


def kernel(embeddings, memberships, subjects, objects, class_V, class_W,
           sub_Vs, sub_Vo, sub_Ws, sub_Wo, sub_w,
           obj_Vs, obj_Vo, obj_Ws, obj_Wo, obj_w):
    e = jnp.pad(embeddings, ((0, NP_ - N), (0, 0)))
    m_pad = jnp.pad(memberships, ((0, NP_ - N), (0, 0)))
    npad = (NENT - 2 * BR) // 2
    pad_s = N + (jnp.arange(npad, dtype=jnp.int32) % (NP_ - N))
    pad_o = N + ((jnp.arange(npad, dtype=jnp.int32) + npad) % (NP_ - N))
    idx_all = jnp.concatenate(
        [subjects, jnp.tile(pad_s, (G_, 1)),
         objects, jnp.tile(pad_o, (G_, 1))], axis=1)
    idx3d = idx_all.reshape(G_, NW, NENT // ICH // NW, ICH)
    idx16 = idx_all.reshape(G_, 16, NCH, ICH)
    zeros = jnp.zeros((ICH, D), jnp.float32)

    for _ in range(ITERS):
        e = _class_update(e, m_pad, class_V, class_W)
        e_ref = jax.new_ref(e)
        rows = _sc_gather(e_ref, idx3d[0])
        for g in range(G_):
            upd = _rel_compute(
                rows,
                sub_Vs[g], sub_Vo[g], sub_Ws[g], sub_Wo[g], sub_w[g],
                obj_Vs[g], obj_Vo[g], obj_Ws[g], obj_Wo[g], obj_w[g]
            ).reshape(NENT, D)
            accv = _sc_scatter_acc(upd, idx16[g], zeros)
            un = _norm_add(rows, accv)
            _sc_scatter_rows(e_ref, idx3d[g], un)
            if g < G_ - 1:
                rows = _sc_gather(e_ref, idx3d[g + 1])
        e = e_ref[...]
    return e[:N]


# speedup vs baseline: 1.7045x; 1.1291x over previous
import functools

import jax
import jax.numpy as jnp
from jax import lax
from jax.experimental import pallas as pl
from jax.experimental.pallas import tpu as pltpu
from jax.experimental.pallas import tpu_sc as plsc

N = 50000
D = 128
K = 16
R = 4
G_ = 2 * R
BR = 10000
ITERS = 2

NP_ = 50176
NENT = 20480
NW = 32
EPW = NENT // NW
ICH = 128

CLS_BLK = 1024
REL_BLK = 1000


def _class_update_body(e_ref, m_ref, va_ref, vb_ref, wa_ref, wb_ref, o_ref):
    e = e_ref[...]
    m = m_ref[...]
    dn = (((1,), (1,)), ((), ()))
    gate_pre = (jax.lax.dot_general(e, va_ref[...], dn)
                + jax.lax.dot_general(m, vb_ref[...], dn))
    dir_pre = (jax.lax.dot_general(e, wa_ref[...], dn)
               + jax.lax.dot_general(m, wb_ref[...], dn))
    x = e + jax.nn.sigmoid(gate_pre) * jnp.maximum(dir_pre, 0.0)
    n = jnp.sqrt(jnp.sum(x * x, axis=1, keepdims=True))
    o_ref[...] = x / jnp.maximum(n, 1e-12)


def _class_update(e, m, class_V, class_W):
    va, vb = class_V[:, :D], class_V[:, D:]
    wa, wb = class_W[:, :D], class_W[:, D:]
    full = lambda s: pl.BlockSpec(s, lambda i: (0, 0))
    return pl.pallas_call(
        _class_update_body,
        grid=(NP_ // CLS_BLK,),
        in_specs=[
            pl.BlockSpec((CLS_BLK, D), lambda i: (i, 0)),
            pl.BlockSpec((CLS_BLK, K), lambda i: (i, 0)),
            full((D, D)), full((D, K)), full((D, D)), full((D, K)),
        ],
        out_specs=pl.BlockSpec((CLS_BLK, D), lambda i: (i, 0)),
        out_shape=jax.ShapeDtypeStruct((NP_, D), jnp.float32),
    )(e, m, va, vb, wa, wb)


def _rel_body(es_ref, eo_ref, svs_ref, svo_ref, sws_ref, swo_ref, sw_ref,
              ovs_ref, ovo_ref, ows_ref, owo_ref, ow_ref, o_ref):
    es = es_ref[...]
    eo = eo_ref[...]
    dn = (((1,), (1,)), ((), ()))
    dg = jax.lax.dot_general
    gs = jax.nn.sigmoid(dg(es, svs_ref[...], dn) + dg(eo, svo_ref[...], dn))
    dot_s = jnp.sum(eo * sw_ref[...], axis=1, keepdims=True)
    dir_s = jnp.maximum(dg(es, sws_ref[...], dn) + dg(eo, swo_ref[...], dn)
                        + es * dot_s, 0.0)
    o_ref[0] = gs * dir_s
    go = jax.nn.sigmoid(dg(es, ovs_ref[...], dn) + dg(eo, ovo_ref[...], dn))
    dot_o = jnp.sum(eo * ow_ref[...], axis=1, keepdims=True)
    dir_o = jnp.maximum(dg(es, ows_ref[...], dn) + dg(eo, owo_ref[...], dn)
                        + es * dot_o, 0.0)
    o_ref[1] = go * dir_o


HALF = NENT // 2
RBLK = 1024


def _rel_compute(rows, svs, svo, sws, swo, sw, ovs, ovo, ows, owo, ow):
    full = lambda: pl.BlockSpec((D, D), lambda i: (0, 0))
    vec = lambda: pl.BlockSpec((1, D), lambda i: (0, 0))
    nblk = HALF // RBLK
    return pl.pallas_call(
        _rel_body,
        grid=(nblk,),
        in_specs=[pl.BlockSpec((RBLK, D), lambda i: (i, 0)),
                  pl.BlockSpec((RBLK, D), lambda i, n=nblk: (i + n, 0)),
                  full(), full(), full(), full(), vec(),
                  full(), full(), full(), full(), vec()],
        out_specs=pl.BlockSpec((2, RBLK, D), lambda i: (0, i, 0)),
        out_shape=jax.ShapeDtypeStruct((2, HALF, D), jnp.float32),
    )(rows, rows, svs, svo, sws, swo, sw.reshape(1, D),
      ovs, ovo, ows, owo, ow.reshape(1, D))


def _norm_add_body(r_ref, a_ref, o_ref):
    x = r_ref[...] + a_ref[...]
    n = jnp.sqrt(jnp.sum(x * x, axis=1, keepdims=True))
    o_ref[...] = x / jnp.maximum(n, 1e-12)


def _norm_add(rows, accv):
    nb = 1024
    blk = pl.BlockSpec((nb, D), lambda i: (i, 0))
    return pl.pallas_call(
        _norm_add_body,
        grid=(NENT // nb,),
        in_specs=[blk, blk],
        out_specs=blk,
        out_shape=jax.ShapeDtypeStruct((NENT, D), jnp.float32),
    )(rows, accv)


def _sc_gather(e, idx3d):
    kpw = NENT // ICH // NW
    mesh = plsc.VectorSubcoreMesh(core_axis_name="c", subcore_axis_name="s")

    @functools.partial(
        pl.kernel, mesh=mesh,
        out_type=jax.ShapeDtypeStruct((NENT, D), jnp.float32),
        scratch_types=[
            pltpu.VMEM((kpw, ICH), jnp.int32),
            pltpu.VMEM((EPW, D), jnp.float32),
            pltpu.SemaphoreType.DMA,
        ],
        name="sc_gather",
    )
    def k(e_hbm, idx_hbm, out_hbm, idx_v, rows_v, sem):
        wid = lax.axis_index("s") * 2 + lax.axis_index("c")
        pltpu.sync_copy(idx_hbm.at[wid], idx_v)
        cps = []
        for j in range(kpw):
            cps.append(pltpu.async_copy(
                e_hbm.at[idx_v.at[j]],
                rows_v.at[pl.ds(j * ICH, ICH)], sem))
        for cp in cps:
            cp.wait()
        pltpu.sync_copy(rows_v, out_hbm.at[pl.ds(wid * EPW, EPW)])

    return k(e, idx3d)


QROWS = 8384
NPASS = 3
GSLOT = 64
ACC_ROWS = QROWS + GSLOT
NCH = NENT // 16 // ICH


def _sc_scatter_acc(upd, idx16, zeros):
    mesh = plsc.VectorSubcoreMesh(core_axis_name="c", subcore_axis_name="s")

    @functools.partial(
        pl.kernel, mesh=mesh,
        out_type=jax.ShapeDtypeStruct((NENT + GSLOT, D), jnp.float32),
        scratch_types=[
            pltpu.VMEM((NCH, ICH), jnp.int32),
            pltpu.VMEM((NCH, ICH), jnp.int32),
            pltpu.VMEM((NCH, ICH), jnp.int32),
            pltpu.VMEM((2, ICH, D), jnp.float32),
            pltpu.VMEM((ICH, D), jnp.float32),
            pltpu.VMEM_SHARED((ACC_ROWS, D), jnp.float32),
            pltpu.SemaphoreType.DMA,
            pltpu.SemaphoreType.DMA,
        ],
        name="sc_scatter_acc",
    )
    def k(upd_hbm, idx_hbm, z_hbm, accv_hbm, idx_v, aidx_v, uidx_v,
          buf_v, zeros_v, acc, sem_ld, sem_st):
        c = lax.axis_index("c")
        s = lax.axis_index("s")
        lane = lax.iota(jnp.int32, 16)
        pltpu.sync_copy(idx_hbm.at[s], idx_v)
        pltpu.sync_copy(z_hbm, zeros_v)
        stripe = ACC_ROWS // 16
        nfull = stripe // ICH
        rem = stripe - nfull * ICH

        def zero_stripe():
            zs = [pltpu.async_copy(
                zeros_v, acc.at[pl.ds(s * stripe + j * ICH, ICH)], sem_st)
                for j in range(nfull)]
            if rem:
                zs.append(pltpu.async_copy(
                    zeros_v.at[pl.ds(0, rem)],
                    acc.at[pl.ds(s * stripe + nfull * ICH, rem)], sem_st))
            for z in zs:
                z.wait()

        zero_stripe()
        plsc.subcore_barrier()
        for p in range(NPASS):
            qb = (2 * p) * QROWS + c * QROWS

            @pl.loop(0, NCH * 8)
            def _(i):
                kk = i >> 3
                sl = pl.ds((i & 7) * 16, 16)
                iv = idx_v[kk, sl]
                rel = iv - qb
                inq = (rel >= 0) & (rel < QROWS)
                spread = (lane + i) & (GSLOT - 1)
                aidx_v[kk, sl] = jnp.where(inq, rel, QROWS + spread)
                uidx_v[kk, sl] = jnp.where(
                    inq, s * (NCH * ICH) + i * 16 + lane, NENT + spread)
            lds = [None] * NCH
            sts = []
            off0 = pl.multiple_of(s * NCH * ICH, ICH)
            lds[0] = pltpu.async_copy(upd_hbm.at[pl.ds(off0, ICH)],
                                      buf_v.at[0], sem_ld)
            for kk in range(NCH):
                if kk + 1 < NCH:
                    if kk >= 1:
                        sts[kk - 1].wait()
                    off = pl.multiple_of((s * NCH + kk + 1) * ICH, ICH)
                    lds[kk + 1] = pltpu.async_copy(
                        upd_hbm.at[pl.ds(off, ICH)],
                        buf_v.at[(kk + 1) % 2], sem_ld)
                lds[kk].wait()
                sts.append(pltpu.async_copy(
                    buf_v.at[kk % 2], acc.at[aidx_v.at[kk]], sem_st, add=True))
            sts[-2].wait()
            sts[-1].wait()
            plsc.subcore_barrier()
            gts = [None] * NCH
            sts = []
            gts[0] = pltpu.async_copy(acc.at[aidx_v.at[0]], buf_v.at[0], sem_ld)
            for kk in range(NCH):
                if kk + 1 < NCH:
                    if kk >= 1:
                        sts[kk - 1].wait()
                    gts[kk + 1] = pltpu.async_copy(
                        acc.at[aidx_v.at[kk + 1]], buf_v.at[(kk + 1) % 2],
                        sem_ld)
                gts[kk].wait()
                sts.append(pltpu.async_copy(
                    buf_v.at[kk % 2], accv_hbm.at[uidx_v.at[kk]], sem_st))
            sts[-2].wait()
            sts[-1].wait()
            plsc.subcore_barrier()
            if p < NPASS - 1:
                zero_stripe()
                plsc.subcore_barrier()

    return k(upd, idx16, zeros)


def _sc_scatter_rows(e_ref, idx3d_g, un):
    kpw = NENT // ICH // NW
    mesh = plsc.VectorSubcoreMesh(core_axis_name="c", subcore_axis_name="s")

    @functools.partial(
        pl.kernel, mesh=mesh, out_type=(),
        scratch_types=[
            pltpu.VMEM((kpw, ICH), jnp.int32),
            pltpu.VMEM((2, ICH, D), jnp.float32),
            pltpu.SemaphoreType.DMA,
        ],
        name="sc_scatter_rows",
    )
    def k(e_hbm, idx_hbm, un_hbm, idx_v, un_v, sem):
        wid = lax.axis_index("s") * 2 + lax.axis_index("c")
        pltpu.sync_copy(idx_hbm.at[wid], idx_v)
        cps = []
        for j in range(kpw):
            if j >= 2:
                cps[j - 2].wait()
            off = pl.multiple_of((wid * kpw + j) * ICH, ICH)
            pltpu.sync_copy(un_hbm.at[pl.ds(off, ICH)], un_v.at[j % 2])
            cps.append(pltpu.async_copy(un_v.at[j % 2],
                                        e_hbm.at[idx_v.at[j]], sem))
        for cp in cps[-2:]:
            cp.wait()

    k(e_ref, idx3d_g, un)


def kernel(embeddings, memberships, subjects, objects, class_V, class_W,
           sub_Vs, sub_Vo, sub_Ws, sub_Wo, sub_w,
           obj_Vs, obj_Vo, obj_Ws, obj_Wo, obj_w):
    e = jnp.pad(embeddings, ((0, NP_ - N), (0, 0)))
    m_pad = jnp.pad(memberships, ((0, NP_ - N), (0, 0)))
    npad = (NENT - 2 * BR) // 2
    pad_s = N + (jnp.arange(npad, dtype=jnp.int32) % (NP_ - N))
    pad_o = N + ((jnp.arange(npad, dtype=jnp.int32) + npad) % (NP_ - N))
    idx_all = jnp.concatenate(
        [subjects, jnp.tile(pad_s, (G_, 1)),
         objects, jnp.tile(pad_o, (G_, 1))], axis=1)
    idx3d = idx_all.reshape(G_, NW, NENT // ICH // NW, ICH)
    idx16 = idx_all.reshape(G_, 16, NCH, ICH)
    zeros = jnp.zeros((ICH, D), jnp.float32)

    for _ in range(ITERS):
        e = _class_update(e, m_pad, class_V, class_W)
        e_ref = jax.new_ref(e)
        rows = _sc_gather(e_ref, idx3d[0])
        for g in range(G_):
            upd = _rel_compute(
                rows,
                sub_Vs[g], sub_Vo[g], sub_Ws[g], sub_Wo[g], sub_w[g],
                obj_Vs[g], obj_Vo[g], obj_Ws[g], obj_Wo[g], obj_w[g]
            ).reshape(NENT, D)
            accv = _sc_scatter_acc(upd, idx16[g], zeros)
            un = _norm_add(rows, accv)
            _sc_scatter_rows(e_ref, idx3d[g], un)
            if g < G_ - 1:
                rows = _sc_gather(e_ref, idx3d[g + 1])
        e = e_ref[...]
    return e[:N]
